# Initial kernel scaffold; baseline (speedup 1.0000x reference)
import jax
import jax.numpy as jnp
from jax.experimental import pallas as pl


def kernel(x, adjacency, Wg0, bg0, gamma0, beta0, Wg1, bg1, gamma1, beta1, Wg2, bg2, gamma2, beta2, Wr0, br0, Wf, bf, Wfr, bfr):
    raise NotImplementedError("write your pallas kernel here")


# speedup vs baseline: 120.4667x; 120.4667x over previous
import jax
import jax.numpy as jnp
from jax.experimental import pallas as pl
from jax.experimental.pallas import tpu as pltpu

_N = 2000
_NP = 2048
_F = 128
_H = 64
_OUT = 128
_BT = 48
_R = 4
_G = _BT // _R
_TS = 256
_EPS = 1e-5
_GINV = 1.0 / float(jnp.sqrt(jnp.float32(1.0 + _EPS)))


def _prep_kernel(a_ref, s_ref, dinv_scr):
    i = pl.program_id(0)

    @pl.when(i == 0)
    def _():
        deg = jnp.sum(a_ref[...], axis=0, keepdims=True) + 1.0
        dinv_scr[...] = jax.lax.rsqrt(deg)

    dinv_row = dinv_scr[...]
    a_tile = a_ref[pl.ds(i * _TS, _TS), :]
    rows = jax.lax.broadcasted_iota(jnp.int32, (_TS, _NP), 0)
    cols = jax.lax.broadcasted_iota(jnp.int32, (_TS, _NP), 1)
    p = (cols == rows + i * _TS).astype(jnp.float32)
    dinv_col = jax.lax.dot_general(p, dinv_row, (((1,), (1,)), ((), ())),
                                   preferred_element_type=jnp.float32)
    s_ref[...] = dinv_col * a_tile * dinv_row + p * (dinv_row * dinv_row)


def _main_kernel(s_ref, x_ref, wg0_ref, wg1_ref, wg2_ref, wr0_ref, wf_ref,
                 wfr_ref, bg0_ref, bg1_ref, bg2_ref, sc0_ref, sc1_ref,
                 sc2_ref, sh0_ref, sh1_ref, sh2_ref, br0_ref, bf_ref,
                 bfr_ref, out_ref):
    s = s_ref[...]
    xv = x_ref[0]

    def per_rep(w_ref, h, din):
        w = w_ref[...]
        parts = [jnp.dot(w, h[k * din:(k + 1) * din, :],
                         preferred_element_type=jnp.float32)
                 for k in range(_R)]
        return jnp.concatenate(parts, axis=0)

    def gcn_bn(h, wg_ref, bg_ref, sc_ref, sh_ref, din):
        hl = per_rep(wg_ref, h, din)
        agg = jnp.dot(hl, s, preferred_element_type=jnp.float32)
        return jnp.maximum((agg + bg_ref[...]) * sc_ref[...] + sh_ref[...],
                           0.0)

    h1 = gcn_bn(xv, wg0_ref, bg0_ref, sc0_ref, sh0_ref, _F)
    h1 = h1 + per_rep(wr0_ref, xv, _F) + br0_ref[...]
    h2 = gcn_bn(h1, wg1_ref, bg1_ref, sc1_ref, sh1_ref, _H) + h1
    h3 = gcn_bn(h2, wg2_ref, bg2_ref, sc2_ref, sh2_ref, _H) + h2
    out = per_rep(wf_ref, h3, _H) + bf_ref[...]
    out = out + per_rep(wfr_ref, xv, _F) + bfr_ref[...]
    out_ref[0] = out


def kernel(x, adjacency, Wg0, bg0, gamma0, beta0, Wg1, bg1, gamma1, beta1,
           Wg2, bg2, gamma2, beta2, Wr0, br0, Wf, bf, Wfr, bfr):
    f32 = jnp.float32

    a_pad = jnp.zeros((_NP, _NP), f32).at[:_N, :_N].set(adjacency)
    s = pl.pallas_call(
        _prep_kernel,
        grid=(_NP // _TS,),
        in_specs=[pl.BlockSpec((_NP, _NP), lambda i: (0, 0))],
        out_specs=pl.BlockSpec((_TS, _NP), lambda i: (i, 0)),
        out_shape=jax.ShapeDtypeStruct((_NP, _NP), f32),
        scratch_shapes=[pltpu.VMEM((1, _NP), f32)],
    )(a_pad)

    xr = x.reshape(_BT, _N, _F)
    xt = jnp.transpose(xr, (0, 2, 1))
    xt = jnp.pad(xt, ((0, 0), (0, 0), (0, _NP - _N)))
    xt = xt.reshape(_G, _R * _F, _NP)

    def colvec(v, reps):
        return jnp.tile(v.reshape(-1, 1), (reps, 1)).astype(f32)

    args = (
        s, xt,
        Wg0.T, Wg1.T, Wg2.T, Wr0.T, Wf.T, Wfr.T,
        colvec(bg0, _R), colvec(bg1, _R), colvec(bg2, _R),
        colvec(gamma0 * _GINV, _R), colvec(gamma1 * _GINV, _R),
        colvec(gamma2 * _GINV, _R),
        colvec(beta0, _R), colvec(beta1, _R), colvec(beta2, _R),
        colvec(br0, _R), colvec(bf, _R), colvec(bfr, _R),
    )

    full = lambda arr: pl.BlockSpec(arr.shape, lambda g: (0,) * arr.ndim)
    in_specs = [full(s),
                pl.BlockSpec((1, _R * _F, _NP), lambda g: (g, 0, 0))]
    in_specs += [full(a) for a in args[2:]]

    out = pl.pallas_call(
        _main_kernel,
        grid=(_G,),
        in_specs=in_specs,
        out_specs=pl.BlockSpec((1, _R * _OUT, _NP), lambda g: (g, 0, 0)),
        out_shape=jax.ShapeDtypeStruct((_G, _R * _OUT, _NP), f32),
    )(*args)

    out = out.reshape(_BT, _OUT, _NP)[:, :, :_N]
    return jnp.transpose(out, (0, 2, 1)).reshape(x.shape[0], x.shape[1],
                                                 _N, _OUT)


# speedup vs baseline: 144.9777x; 1.2035x over previous
import math

import jax
import jax.numpy as jnp
from jax.experimental import pallas as pl
from jax.experimental.pallas import tpu as pltpu

_N = 2000
_NP = 2048
_F = 128
_H = 64
_OUT = 128
_BT = 48
_R = 4
_G = _BT // _R
_TS = 256
_EPS = 1e-5
_GINV = 1.0 / math.sqrt(1.0 + _EPS)


def _prep_kernel(a_full_ref, a_col_ref, ahat_ref, dinv_scr):
    i = pl.program_id(0)

    @pl.when(i == 0)
    def _():
        deg = jnp.sum(a_full_ref[...], axis=0, keepdims=True) + 1.0
        dinv_scr[...] = jax.lax.rsqrt(deg)

    dinv_row = dinv_scr[...]
    at = jnp.transpose(a_col_ref[...])
    rows = jax.lax.broadcasted_iota(jnp.int32, (_TS, _NP), 0)
    cols = jax.lax.broadcasted_iota(jnp.int32, (_TS, _NP), 1)
    p = (cols == rows + i * _TS).astype(jnp.float32)
    dinv_col = jax.lax.dot_general(p, dinv_row, (((1,), (1,)), ((), ())),
                                   preferred_element_type=jnp.float32)
    ahat_ref[...] = dinv_col * at * dinv_row + p * (dinv_row * dinv_row)


def _main_kernel(ahat_ref, x_ref, wg0_ref, wr0_ref, wfr_ref, wg1b_ref,
                 wg2b_ref, wfb_ref, bg0_ref, bg1_ref, bg2_ref, sc0_ref,
                 sc1_ref, sc2_ref, sh0_ref, sh1_ref, sh2_ref, br0_ref,
                 bf_ref, bfr_ref, out_ref):
    ahat = ahat_ref[...]
    f32 = jnp.float32

    def from_x(w_ref):
        w = w_ref[...]
        cols = [jnp.dot(x_ref[k], w, preferred_element_type=f32)
                for k in range(_R)]
        stacked = jnp.concatenate(cols, axis=1)
        return jnp.concatenate(
            [stacked, jnp.zeros((_NP - _N, stacked.shape[1]), f32)], axis=0)

    def bn_relu(agg, bg_ref, sc_ref, sh_ref):
        return jnp.maximum((agg + bg_ref[...]) * sc_ref[...] + sh_ref[...],
                           0.0)

    hl0 = from_x(wg0_ref)
    h1 = bn_relu(jnp.dot(ahat, hl0, preferred_element_type=f32),
                 bg0_ref, sc0_ref, sh0_ref)
    h1 = h1 + from_x(wr0_ref) + br0_ref[...]
    hl1 = jnp.dot(h1, wg1b_ref[...], preferred_element_type=f32)
    h2 = bn_relu(jnp.dot(ahat, hl1, preferred_element_type=f32),
                 bg1_ref, sc1_ref, sh1_ref) + h1
    hl2 = jnp.dot(h2, wg2b_ref[...], preferred_element_type=f32)
    h3 = bn_relu(jnp.dot(ahat, hl2, preferred_element_type=f32),
                 bg2_ref, sc2_ref, sh2_ref) + h2
    out = jnp.dot(h3, wfb_ref[...], preferred_element_type=f32) + bf_ref[...]
    out = out[:_N, :] + from_x(wfr_ref)[:_N, :] + bfr_ref[...]
    for k in range(_R):
        out_ref[k] = out[:, k * _OUT:(k + 1) * _OUT]


def kernel(x, adjacency, Wg0, bg0, gamma0, beta0, Wg1, bg1, gamma1, beta1,
           Wg2, bg2, gamma2, beta2, Wr0, br0, Wf, bf, Wfr, bfr):
    f32 = jnp.float32

    a_pad = jnp.pad(adjacency, ((0, _NP - _N), (0, _NP - _N)))
    ahat = pl.pallas_call(
        _prep_kernel,
        grid=(_NP // _TS,),
        in_specs=[pl.BlockSpec((_NP, _NP), lambda i: (0, 0)),
                  pl.BlockSpec((_NP, _TS), lambda i: (0, i))],
        out_specs=pl.BlockSpec((_TS, _NP), lambda i: (i, 0)),
        out_shape=jax.ShapeDtypeStruct((_NP, _NP), f32),
        scratch_shapes=[pltpu.VMEM((1, _NP), f32)],
    )(a_pad, a_pad)

    xr = x.reshape(_BT, _N, _F)

    def rowvec(v, reps):
        return jnp.tile(v.reshape(1, -1), (1, reps)).astype(f32)

    def blockdiag(w, reps):
        din, dout = w.shape
        eye = jnp.eye(reps, dtype=f32)
        return (eye[:, None, :, None] * w[None, :, None, :]).reshape(
            reps * din, reps * dout)

    args = (
        ahat, xr,
        Wg0, Wr0, Wfr,
        blockdiag(Wg1, _R), blockdiag(Wg2, _R), blockdiag(Wf, _R),
        rowvec(bg0, _R), rowvec(bg1, _R), rowvec(bg2, _R),
        rowvec(gamma0 * _GINV, _R), rowvec(gamma1 * _GINV, _R),
        rowvec(gamma2 * _GINV, _R),
        rowvec(beta0, _R), rowvec(beta1, _R), rowvec(beta2, _R),
        rowvec(br0, _R), rowvec(bf, _R), rowvec(bfr, _R),
    )

    full = lambda arr: pl.BlockSpec(arr.shape, lambda g: (0,) * arr.ndim)
    in_specs = [full(ahat),
                pl.BlockSpec((_R, _N, _F), lambda g: (g, 0, 0))]
    in_specs += [full(a) for a in args[2:]]

    out = pl.pallas_call(
        _main_kernel,
        grid=(_G,),
        in_specs=in_specs,
        out_specs=pl.BlockSpec((_R, _N, _OUT), lambda g: (g, 0, 0)),
        out_shape=jax.ShapeDtypeStruct((_BT, _N, _OUT), f32),
    )(*args)

    return out.reshape(x.shape[0], x.shape[1], _N, _OUT)


# speedup vs baseline: 146.2243x; 1.0086x over previous
import math

import jax
import jax.numpy as jnp
from jax.experimental import pallas as pl
from jax.experimental.pallas import tpu as pltpu

_N = 2000
_NP = 2048
_F = 128
_H = 64
_OUT = 128
_BT = 48
_R = 4
_G = _BT // _R
_TS = 256
_EPS = 1e-5
_GINV = 1.0 / math.sqrt(1.0 + _EPS)


def _prep_kernel(a_full_ref, a_col_ref, ahat_ref, dinv_scr):
    i = pl.program_id(0)

    @pl.when(i == 0)
    def _():
        deg = jnp.sum(a_full_ref[...], axis=0, keepdims=True) + 1.0
        dinv_scr[...] = jax.lax.rsqrt(deg)

    dinv_row = dinv_scr[...]
    at = jnp.transpose(a_col_ref[...])
    rows = jax.lax.broadcasted_iota(jnp.int32, (_TS, _NP), 0)
    cols = jax.lax.broadcasted_iota(jnp.int32, (_TS, _NP), 1)
    p = (cols == rows + i * _TS).astype(jnp.float32)
    dinv_col = jax.lax.dot_general(p, dinv_row, (((1,), (1,)), ((), ())),
                                   preferred_element_type=jnp.float32)
    ahat = dinv_col * at * dinv_row + p * (dinv_row * dinv_row)
    ahat_ref[...] = ahat.astype(jnp.bfloat16)


def _main_kernel(ahat_ref, x_ref, wg0_ref, wr0_ref, wfr_ref, wg1b_ref,
                 wg2b_ref, wfb_ref, bg0_ref, bg1_ref, bg2_ref, sc0_ref,
                 sc1_ref, sc2_ref, sh0_ref, sh1_ref, sh2_ref, br0_ref,
                 bf_ref, bfr_ref, out_ref):
    ahat = ahat_ref[...]
    f32 = jnp.float32

    def agg(hl):
        return jnp.dot(ahat, hl.astype(jnp.bfloat16),
                       preferred_element_type=f32)

    def from_x(w_ref):
        w = w_ref[...]
        cols = [jnp.dot(x_ref[k], w, preferred_element_type=f32)
                for k in range(_R)]
        stacked = jnp.concatenate(cols, axis=1)
        return jnp.concatenate(
            [stacked, jnp.zeros((_NP - _N, stacked.shape[1]), f32)], axis=0)

    def bn_relu(agg, bg_ref, sc_ref, sh_ref):
        return jnp.maximum((agg + bg_ref[...]) * sc_ref[...] + sh_ref[...],
                           0.0)

    hl0 = from_x(wg0_ref)
    h1 = bn_relu(agg(hl0), bg0_ref, sc0_ref, sh0_ref)
    h1 = h1 + from_x(wr0_ref) + br0_ref[...]
    hl1 = jnp.dot(h1, wg1b_ref[...], preferred_element_type=f32)
    h2 = bn_relu(agg(hl1), bg1_ref, sc1_ref, sh1_ref) + h1
    hl2 = jnp.dot(h2, wg2b_ref[...], preferred_element_type=f32)
    h3 = bn_relu(agg(hl2), bg2_ref, sc2_ref, sh2_ref) + h2
    out = jnp.dot(h3, wfb_ref[...], preferred_element_type=f32) + bf_ref[...]
    out = out[:_N, :] + from_x(wfr_ref)[:_N, :] + bfr_ref[...]
    for k in range(_R):
        out_ref[k] = out[:, k * _OUT:(k + 1) * _OUT]


def kernel(x, adjacency, Wg0, bg0, gamma0, beta0, Wg1, bg1, gamma1, beta1,
           Wg2, bg2, gamma2, beta2, Wr0, br0, Wf, bf, Wfr, bfr):
    f32 = jnp.float32

    a_pad = jnp.pad(adjacency, ((0, _NP - _N), (0, _NP - _N)))
    ahat = pl.pallas_call(
        _prep_kernel,
        grid=(_NP // _TS,),
        in_specs=[pl.BlockSpec((_NP, _NP), lambda i: (0, 0)),
                  pl.BlockSpec((_NP, _TS), lambda i: (0, i))],
        out_specs=pl.BlockSpec((_TS, _NP), lambda i: (i, 0)),
        out_shape=jax.ShapeDtypeStruct((_NP, _NP), jnp.bfloat16),
        scratch_shapes=[pltpu.VMEM((1, _NP), f32)],
    )(a_pad, a_pad)

    xr = x.reshape(_BT, _N, _F)

    def rowvec(v, reps):
        return jnp.tile(v.reshape(1, -1), (1, reps)).astype(f32)

    def blockdiag(w, reps):
        din, dout = w.shape
        eye = jnp.eye(reps, dtype=f32)
        return (eye[:, None, :, None] * w[None, :, None, :]).reshape(
            reps * din, reps * dout)

    args = (
        ahat, xr,
        Wg0, Wr0, Wfr,
        blockdiag(Wg1, _R), blockdiag(Wg2, _R), blockdiag(Wf, _R),
        rowvec(bg0, _R), rowvec(bg1, _R), rowvec(bg2, _R),
        rowvec(gamma0 * _GINV, _R), rowvec(gamma1 * _GINV, _R),
        rowvec(gamma2 * _GINV, _R),
        rowvec(beta0, _R), rowvec(beta1, _R), rowvec(beta2, _R),
        rowvec(br0, _R), rowvec(bf, _R), rowvec(bfr, _R),
    )

    full = lambda arr: pl.BlockSpec(arr.shape, lambda g: (0,) * arr.ndim)
    in_specs = [full(ahat),
                pl.BlockSpec((_R, _N, _F), lambda g: (g, 0, 0))]
    in_specs += [full(a) for a in args[2:]]

    out = pl.pallas_call(
        _main_kernel,
        grid=(_G,),
        in_specs=in_specs,
        out_specs=pl.BlockSpec((_R, _N, _OUT), lambda g: (g, 0, 0)),
        out_shape=jax.ShapeDtypeStruct((_BT, _N, _OUT), f32),
    )(*args)

    return out.reshape(x.shape[0], x.shape[1], _N, _OUT)


# speedup vs baseline: 158.1164x; 1.0813x over previous
import math

import jax
import jax.numpy as jnp
from jax.experimental import pallas as pl
from jax.experimental.pallas import tpu as pltpu

_N = 2000
_NP = 2048
_F = 128
_H = 64
_OUT = 128
_BT = 48
_R = 4
_G = _BT // _R
_TS = 256
_EPS = 1e-5
_GINV = 1.0 / math.sqrt(1.0 + _EPS)


def _prep_kernel(a_full_ref, a_col_ref, ahat_ref, dinv_scr):
    i = pl.program_id(0)

    @pl.when(i == 0)
    def _():
        deg = jnp.sum(a_full_ref[...], axis=0, keepdims=True) + 1.0
        dinv_scr[...] = jax.lax.rsqrt(deg)

    dinv_row = dinv_scr[...]
    at = jnp.transpose(a_col_ref[...])
    rows = jax.lax.broadcasted_iota(jnp.int32, (_TS, _NP), 0)
    cols = jax.lax.broadcasted_iota(jnp.int32, (_TS, _NP), 1)
    p = (cols == rows + i * _TS).astype(jnp.float32)
    dinv_col = jax.lax.dot_general(p, dinv_row, (((1,), (1,)), ((), ())),
                                   preferred_element_type=jnp.float32)
    ahat_ref[...] = dinv_col * at * dinv_row + p * (dinv_row * dinv_row)


def _tile_lanes(row, reps):
    return jnp.concatenate([row] * reps, axis=1)


def _blockdiag(w, reps):
    din, dout = w.shape
    tiled = jnp.concatenate(
        [jnp.concatenate([w] * reps, axis=1)] * reps, axis=0)
    r = jax.lax.broadcasted_iota(jnp.int32, (reps * din, reps * dout), 0)
    c = jax.lax.broadcasted_iota(jnp.int32, (reps * din, reps * dout), 1)
    return jnp.where((r // din) == (c // dout), tiled, 0.0)


def _main_kernel(ahat_ref, x_ref, wg0_ref, wg1_ref, wg2_ref, wr0_ref,
                 wf_ref, wfr_ref, bg0_ref, bg1_ref, bg2_ref, ga0_ref,
                 ga1_ref, ga2_ref, be0_ref, be1_ref, be2_ref, br0_ref,
                 bf_ref, bfr_ref, out_ref):
    ahat = ahat_ref[...]
    f32 = jnp.float32

    def from_x(w_ref):
        w = w_ref[...]
        cols = [jnp.dot(x_ref[k], w, preferred_element_type=f32)
                for k in range(_R)]
        stacked = jnp.concatenate(cols, axis=1)
        return jnp.concatenate(
            [stacked, jnp.zeros((_NP - _N, stacked.shape[1]), f32)], axis=0)

    def bn_relu(v, bg_ref, ga_ref, be_ref):
        bg = _tile_lanes(bg_ref[...], _R)
        sc = _tile_lanes(ga_ref[...] * _GINV, _R)
        sh = _tile_lanes(be_ref[...], _R)
        return jnp.maximum((v + bg) * sc + sh, 0.0)

    hl0 = from_x(wg0_ref)
    h1 = bn_relu(jnp.dot(ahat, hl0, preferred_element_type=f32),
                 bg0_ref, ga0_ref, be0_ref)
    h1 = h1 + from_x(wr0_ref) + _tile_lanes(br0_ref[...], _R)
    hl1 = jnp.dot(h1, _blockdiag(wg1_ref[...], _R),
                  preferred_element_type=f32)
    h2 = bn_relu(jnp.dot(ahat, hl1, preferred_element_type=f32),
                 bg1_ref, ga1_ref, be1_ref) + h1
    hl2 = jnp.dot(h2, _blockdiag(wg2_ref[...], _R),
                  preferred_element_type=f32)
    h3 = bn_relu(jnp.dot(ahat, hl2, preferred_element_type=f32),
                 bg2_ref, ga2_ref, be2_ref) + h2
    out = (jnp.dot(h3, _blockdiag(wf_ref[...], _R),
                   preferred_element_type=f32)
           + _tile_lanes(bf_ref[...], _R))
    out = (out[:_N, :] + from_x(wfr_ref)[:_N, :]
           + _tile_lanes(bfr_ref[...], _R))
    for k in range(_R):
        out_ref[k] = out[:, k * _OUT:(k + 1) * _OUT]


def kernel(x, adjacency, Wg0, bg0, gamma0, beta0, Wg1, bg1, gamma1, beta1,
           Wg2, bg2, gamma2, beta2, Wr0, br0, Wf, bf, Wfr, bfr):
    f32 = jnp.float32

    a_pad = jnp.pad(adjacency, ((0, _NP - _N), (0, _NP - _N)))
    ahat = pl.pallas_call(
        _prep_kernel,
        grid=(_NP // _TS,),
        in_specs=[pl.BlockSpec((_NP, _NP), lambda i: (0, 0)),
                  pl.BlockSpec((_NP, _TS), lambda i: (0, i))],
        out_specs=pl.BlockSpec((_TS, _NP), lambda i: (i, 0)),
        out_shape=jax.ShapeDtypeStruct((_NP, _NP), f32),
        scratch_shapes=[pltpu.VMEM((1, _NP), f32)],
    )(a_pad, a_pad)

    xr = x.reshape(_BT, _N, _F)

    row = lambda v: v.reshape(1, -1)
    args = (
        ahat, xr,
        Wg0, Wg1, Wg2, Wr0, Wf, Wfr,
        row(bg0), row(bg1), row(bg2),
        row(gamma0), row(gamma1), row(gamma2),
        row(beta0), row(beta1), row(beta2),
        row(br0), row(bf), row(bfr),
    )

    full = lambda arr: pl.BlockSpec(arr.shape, lambda g: (0,) * arr.ndim)
    in_specs = [full(ahat),
                pl.BlockSpec((_R, _N, _F), lambda g: (g, 0, 0))]
    in_specs += [full(a) for a in args[2:]]

    out = pl.pallas_call(
        _main_kernel,
        grid=(_G,),
        in_specs=in_specs,
        out_specs=pl.BlockSpec((_R, _N, _OUT), lambda g: (g, 0, 0)),
        out_shape=jax.ShapeDtypeStruct((_BT, _N, _OUT), f32),
    )(*args)

    return out.reshape(x.shape[0], x.shape[1], _N, _OUT)


# speedup vs baseline: 173.3024x; 1.0960x over previous
import math

import jax
import jax.numpy as jnp
from jax.experimental import pallas as pl
from jax.experimental.pallas import tpu as pltpu

_N = 2000
_F = 128
_H = 64
_OUT = 128
_BT = 48
_R = 4
_G = _BT // _R
_EPS = 1e-5
_GINV = 1.0 / math.sqrt(1.0 + _EPS)


def _tile_lanes(row, reps):
    return jnp.concatenate([row] * reps, axis=1)


def _blockdiag(w, reps):
    din, dout = w.shape
    tiled = jnp.concatenate(
        [jnp.concatenate([w] * reps, axis=1)] * reps, axis=0)
    r = jax.lax.broadcasted_iota(jnp.int32, (reps * din, reps * dout), 0)
    c = jax.lax.broadcasted_iota(jnp.int32, (reps * din, reps * dout), 1)
    return jnp.where((r // din) == (c // dout), tiled, 0.0)


_TN = (((0,), (0,)), ((), ()))


def _main_kernel(a_ref, x_ref, wg0_ref, wg1_ref, wg2_ref, wr0_ref,
                 wf_ref, wfr_ref, bg0_ref, bg1_ref, bg2_ref, ga0_ref,
                 ga1_ref, ga2_ref, be0_ref, be1_ref, be2_ref, br0_ref,
                 bf_ref, bfr_ref, out_ref,
                 dinv_scr, wbd1_scr, wbd2_scr):
    f32 = jnp.float32
    g = pl.program_id(0)

    @pl.when(g == 0)
    def _():
        ones = jnp.ones((_N, 1), jnp.bfloat16)
        deg = jax.lax.dot_general(a_ref[...], ones, _TN,
                                  preferred_element_type=f32) + 1.0
        dinv_scr[...] = jax.lax.rsqrt(deg)
        wbd1_scr[...] = _blockdiag(wg1_ref[...], _R)
        wbd2_scr[...] = _blockdiag(wg2_ref[...], _R)

    a = a_ref[...]
    dinv = dinv_scr[...]

    def from_x(w_ref):
        w = w_ref[...]
        cols = [jnp.dot(x_ref[k], w, preferred_element_type=f32)
                for k in range(_R)]
        return jnp.concatenate(cols, axis=1)

    def agg(hl):
        z = dinv * hl
        return dinv * (jax.lax.dot_general(a, z.astype(jnp.bfloat16), _TN,
                                           preferred_element_type=f32) + z)

    def bn_relu(v, bg_ref, ga_ref, be_ref):
        bg = _tile_lanes(bg_ref[...], _R)
        sc = _tile_lanes(ga_ref[...] * _GINV, _R)
        sh = _tile_lanes(be_ref[...], _R)
        return jnp.maximum((v + bg) * sc + sh, 0.0)

    h1 = bn_relu(agg(from_x(wg0_ref)), bg0_ref, ga0_ref, be0_ref)
    h1 = h1 + from_x(wr0_ref) + _tile_lanes(br0_ref[...], _R)
    hl1 = jnp.dot(h1, wbd1_scr[...], preferred_element_type=f32)
    h2 = bn_relu(agg(hl1), bg1_ref, ga1_ref, be1_ref) + h1
    hl2 = jnp.dot(h2, wbd2_scr[...], preferred_element_type=f32)
    h3 = bn_relu(agg(hl2), bg2_ref, ga2_ref, be2_ref) + h2
    wf = wf_ref[...]
    wfr = wfr_ref[...]
    for k in range(_R):
        out_ref[k] = (jnp.dot(h3[:, k * _H:(k + 1) * _H], wf,
                              preferred_element_type=f32) + bf_ref[...]
                      + jnp.dot(x_ref[k], wfr,
                                preferred_element_type=f32) + bfr_ref[...])


def kernel(x, adjacency, Wg0, bg0, gamma0, beta0, Wg1, bg1, gamma1, beta1,
           Wg2, bg2, gamma2, beta2, Wr0, br0, Wf, bf, Wfr, bfr):
    f32 = jnp.float32
    xr = x.reshape(_BT, _N, _F)

    row = lambda v: v.reshape(1, -1)
    args = (
        adjacency.astype(jnp.bfloat16), xr,
        Wg0, Wg1, Wg2, Wr0, Wf, Wfr,
        row(bg0), row(bg1), row(bg2),
        row(gamma0), row(gamma1), row(gamma2),
        row(beta0), row(beta1), row(beta2),
        row(br0), row(bf), row(bfr),
    )

    full = lambda arr: pl.BlockSpec(arr.shape, lambda g: (0,) * arr.ndim)
    in_specs = [full(adjacency),
                pl.BlockSpec((_R, _N, _F), lambda g: (g, 0, 0))]
    in_specs += [full(a) for a in args[2:]]

    out = pl.pallas_call(
        _main_kernel,
        grid=(_G,),
        in_specs=in_specs,
        out_specs=pl.BlockSpec((_R, _N, _OUT), lambda g: (g, 0, 0)),
        out_shape=jax.ShapeDtypeStruct((_BT, _N, _OUT), f32),
        scratch_shapes=[pltpu.VMEM((_N, 1), f32),
                        pltpu.VMEM((_R * _H, _R * _H), f32),
                        pltpu.VMEM((_R * _H, _R * _H), f32)],
    )(*args)

    return out.reshape(x.shape[0], x.shape[1], _N, _OUT)


# speedup vs baseline: 174.5915x; 1.0074x over previous
import math

import jax
import jax.numpy as jnp
from jax.experimental import pallas as pl
from jax.experimental.pallas import tpu as pltpu

_N = 2000
_F = 128
_H = 64
_OUT = 128
_BT = 48
_R = 4
_G = _BT // _R
_EPS = 1e-5
_GINV = 1.0 / math.sqrt(1.0 + _EPS)


def _tile_lanes(row, reps):
    return jnp.concatenate([row] * reps, axis=1)


def _blockdiag(w, reps):
    din, dout = w.shape
    tiled = jnp.concatenate(
        [jnp.concatenate([w] * reps, axis=1)] * reps, axis=0)
    r = jax.lax.broadcasted_iota(jnp.int32, (reps * din, reps * dout), 0)
    c = jax.lax.broadcasted_iota(jnp.int32, (reps * din, reps * dout), 1)
    return jnp.where((r // din) == (c // dout), tiled, 0.0)


_TN = (((0,), (0,)), ((), ()))


def _main_kernel(a_ref, x_ref, wg0_ref, wg1_ref, wg2_ref, wr0_ref,
                 wf_ref, wfr_ref, bg0_ref, bg1_ref, bg2_ref, ga0_ref,
                 ga1_ref, ga2_ref, be0_ref, be1_ref, be2_ref, br0_ref,
                 bf_ref, bfr_ref, out_ref,
                 dinv_scr, cst_scr, wbd1_scr, wbd2_scr):
    f32 = jnp.float32
    g = pl.program_id(0)

    @pl.when(g == 0)
    def _():
        ones = jnp.ones((_N, 1), jnp.bfloat16)
        deg = jax.lax.dot_general(a_ref[...], ones, _TN,
                                  preferred_element_type=f32)
        dinv_scr[...] = jax.lax.rsqrt(deg)
        for i, (bg, ga, be) in enumerate(
                ((bg0_ref, ga0_ref, be0_ref), (bg1_ref, ga1_ref, be1_ref),
                 (bg2_ref, ga2_ref, be2_ref))):
            sc = ga[...] * _GINV
            cst_scr[i:i + 1, :] = _tile_lanes(sc, _R)
            cst_scr[3 + i:4 + i, :] = _tile_lanes(bg[...] * sc + be[...], _R)
        cst_scr[6:7, :] = _tile_lanes(br0_ref[...], _R)
        wbd1_scr[...] = _blockdiag(wg1_ref[...], _R)
        wbd2_scr[...] = _blockdiag(wg2_ref[...], _R)

    a = a_ref[...]
    dinv = dinv_scr[...]

    def from_x(w_ref):
        w = w_ref[...]
        cols = [jnp.dot(x_ref[k], w, preferred_element_type=f32)
                for k in range(_R)]
        return jnp.concatenate(cols, axis=1)

    def gcn_bn_relu(hl, i):
        zb = (dinv * hl).astype(jnp.bfloat16)
        d = jax.lax.dot_general(a, zb, _TN, preferred_element_type=f32)
        return jnp.maximum(d * dinv * cst_scr[i:i + 1, :]
                           + cst_scr[3 + i:4 + i, :], 0.0)

    h1 = gcn_bn_relu(from_x(wg0_ref), 0)
    h1 = h1 + from_x(wr0_ref) + cst_scr[6:7, :]
    hl1 = jnp.dot(h1, wbd1_scr[...], preferred_element_type=f32)
    h2 = gcn_bn_relu(hl1, 1) + h1
    hl2 = jnp.dot(h2, wbd2_scr[...], preferred_element_type=f32)
    h3 = gcn_bn_relu(hl2, 2) + h2
    wf = wf_ref[...]
    wfr = wfr_ref[...]
    for k in range(_R):
        out_ref[k] = (jnp.dot(h3[:, k * _H:(k + 1) * _H], wf,
                              preferred_element_type=f32) + bf_ref[...]
                      + jnp.dot(x_ref[k], wfr,
                                preferred_element_type=f32) + bfr_ref[...])


def kernel(x, adjacency, Wg0, bg0, gamma0, beta0, Wg1, bg1, gamma1, beta1,
           Wg2, bg2, gamma2, beta2, Wr0, br0, Wf, bf, Wfr, bfr):
    f32 = jnp.float32
    xr = x.reshape(_BT, _N, _F)

    row = lambda v: v.reshape(1, -1)
    a_aug = (adjacency + jnp.eye(_N, dtype=f32)).astype(jnp.bfloat16)
    args = (
        a_aug, xr,
        Wg0, Wg1, Wg2, Wr0, Wf, Wfr,
        row(bg0), row(bg1), row(bg2),
        row(gamma0), row(gamma1), row(gamma2),
        row(beta0), row(beta1), row(beta2),
        row(br0), row(bf), row(bfr),
    )

    full = lambda arr: pl.BlockSpec(arr.shape, lambda g: (0,) * arr.ndim)
    in_specs = [full(adjacency),
                pl.BlockSpec((_R, _N, _F), lambda g: (g, 0, 0))]
    in_specs += [full(a) for a in args[2:]]

    out = pl.pallas_call(
        _main_kernel,
        grid=(_G,),
        in_specs=in_specs,
        out_specs=pl.BlockSpec((_R, _N, _OUT), lambda g: (g, 0, 0)),
        out_shape=jax.ShapeDtypeStruct((_BT, _N, _OUT), f32),
        scratch_shapes=[pltpu.VMEM((_N, 1), f32),
                        pltpu.VMEM((8, _R * _H), f32),
                        pltpu.VMEM((_R * _H, _R * _H), f32),
                        pltpu.VMEM((_R * _H, _R * _H), f32)],
    )(*args)

    return out.reshape(x.shape[0], x.shape[1], _N, _OUT)


# speedup vs baseline: 174.6027x; 1.0001x over previous
import math

import jax
import jax.numpy as jnp
from jax.experimental import pallas as pl
from jax.experimental.pallas import tpu as pltpu

_N = 2000
_F = 128
_H = 64
_OUT = 128
_BT = 48
_R = 4
_G = _BT // _R
_EPS = 1e-5
_GINV = 1.0 / math.sqrt(1.0 + _EPS)


def _tile_lanes(row, reps):
    return jnp.concatenate([row] * reps, axis=1)


def _blockdiag(w, reps):
    din, dout = w.shape
    tiled = jnp.concatenate(
        [jnp.concatenate([w] * reps, axis=1)] * reps, axis=0)
    r = jax.lax.broadcasted_iota(jnp.int32, (reps * din, reps * dout), 0)
    c = jax.lax.broadcasted_iota(jnp.int32, (reps * din, reps * dout), 1)
    return jnp.where((r // din) == (c // dout), tiled, 0.0)


_TN = (((0,), (0,)), ((), ()))


def _main_kernel(a_ref, x_ref, wg0_ref, wg1_ref, wg2_ref, wr0_ref,
                 wf_ref, wfr_ref, bg0_ref, bg1_ref, bg2_ref, ga0_ref,
                 ga1_ref, ga2_ref, be0_ref, be1_ref, be2_ref, br0_ref,
                 bf_ref, bfr_ref, out_ref,
                 dinv_scr, cst_scr, wbd1_scr, wbd2_scr):
    f32 = jnp.float32
    g = pl.program_id(0)

    @pl.when(g == 0)
    def _():
        ones = jnp.ones((_N, 1), jnp.bfloat16)
        deg = jax.lax.dot_general(a_ref[...], ones, _TN,
                                  preferred_element_type=f32)
        dinv_scr[...] = jax.lax.rsqrt(deg)
        for i, (bg, ga, be) in enumerate(
                ((bg0_ref, ga0_ref, be0_ref), (bg1_ref, ga1_ref, be1_ref),
                 (bg2_ref, ga2_ref, be2_ref))):
            sc = ga[...] * _GINV
            cst_scr[i:i + 1, :] = _tile_lanes(sc, _R)
            cst_scr[3 + i:4 + i, :] = _tile_lanes(bg[...] * sc + be[...], _R)
        cst_scr[6:7, :] = _tile_lanes(br0_ref[...], _R)
        wbd1_scr[...] = _blockdiag(wg1_ref[...], 4)
        wbd2_scr[...] = _blockdiag(wg2_ref[...], 4)

    a = a_ref[...]
    dinv = dinv_scr[...]

    def from_x(w_ref):
        w = w_ref[...]
        cols = [jnp.dot(x_ref[k], w, preferred_element_type=f32)
                for k in range(_R)]
        return jnp.concatenate(cols, axis=1)

    def gcn_bn_relu(zb, i):
        d = jax.lax.dot_general(a, zb, _TN, preferred_element_type=f32)
        return jnp.maximum(d * dinv * cst_scr[i:i + 1, :]
                           + cst_scr[3 + i:4 + i, :], 0.0)

    def zdot(h, wbd_scr):
        wbd = wbd_scr[...]
        c = 4 * _H
        return jnp.concatenate(
            [(dinv * jnp.dot(h[:, j * c:(j + 1) * c], wbd,
                             preferred_element_type=f32)
              ).astype(jnp.bfloat16) for j in range(_R // 4)],
            axis=1)

    h1 = gcn_bn_relu((dinv * from_x(wg0_ref)).astype(jnp.bfloat16), 0)
    h1 = h1 + from_x(wr0_ref) + cst_scr[6:7, :]
    h2 = gcn_bn_relu(zdot(h1, wbd1_scr), 1) + h1
    h3 = gcn_bn_relu(zdot(h2, wbd2_scr), 2) + h2
    wf = wf_ref[...]
    wfr = wfr_ref[...]
    for k in range(_R):
        out_ref[k] = (jnp.dot(h3[:, k * _H:(k + 1) * _H], wf,
                              preferred_element_type=f32) + bf_ref[...]
                      + jnp.dot(x_ref[k], wfr,
                                preferred_element_type=f32) + bfr_ref[...])


def kernel(x, adjacency, Wg0, bg0, gamma0, beta0, Wg1, bg1, gamma1, beta1,
           Wg2, bg2, gamma2, beta2, Wr0, br0, Wf, bf, Wfr, bfr):
    f32 = jnp.float32
    xr = x.reshape(_BT, _N, _F)

    row = lambda v: v.reshape(1, -1)
    a_aug = (adjacency + jnp.eye(_N, dtype=f32)).astype(jnp.bfloat16)
    args = (
        a_aug, xr,
        Wg0, Wg1, Wg2, Wr0, Wf, Wfr,
        row(bg0), row(bg1), row(bg2),
        row(gamma0), row(gamma1), row(gamma2),
        row(beta0), row(beta1), row(beta2),
        row(br0), row(bf), row(bfr),
    )

    full = lambda arr: pl.BlockSpec(arr.shape, lambda g: (0,) * arr.ndim)
    in_specs = [full(adjacency),
                pl.BlockSpec((_R, _N, _F), lambda g: (g, 0, 0))]
    in_specs += [full(a) for a in args[2:]]

    out = pl.pallas_call(
        _main_kernel,
        grid=(_G,),
        in_specs=in_specs,
        out_specs=pl.BlockSpec((_R, _N, _OUT), lambda g: (g, 0, 0)),
        out_shape=jax.ShapeDtypeStruct((_BT, _N, _OUT), f32),
        scratch_shapes=[pltpu.VMEM((_N, 1), f32),
                        pltpu.VMEM((8, _R * _H), f32),
                        pltpu.VMEM((4 * _H, 4 * _H), f32),
                        pltpu.VMEM((4 * _H, 4 * _H), f32)],
    )(*args)

    return out.reshape(x.shape[0], x.shape[1], _N, _OUT)
